# Initial kernel scaffold; baseline (speedup 1.0000x reference)
import jax
import jax.numpy as jnp
from jax.experimental import pallas as pl


def kernel(x, mask, rows):
    raise NotImplementedError("write your pallas kernel here")


# speedup vs baseline: 1.1993x; 1.1993x over previous
import functools

import jax
import jax.numpy as jnp
from jax import lax
from jax.experimental import pallas as pl
from jax.experimental.pallas import tpu as pltpu
from jax.experimental.pallas import tpu_sc as plsc

_B = 16384
_F = 100
_N = _B * _F

_NC = 2
_NS = 16
_NW = _NC * _NS
_L = 16

_CHUNK = _N // _NW
_W = 2048
_NG = _CHUNK // _W
_GI = 128
_NJ = _W // _GI
_STAGE = _N // _NS


def _body(x_hbm, mask_hbm, rows_hbm, out_hbm,
          xsh, rows_v, mask_v, idx_v, gout, sem):
    cid = lax.axis_index("c")
    sid = lax.axis_index("s")
    wid = sid * _NC + cid

    pltpu.sync_copy(x_hbm.at[pl.ds(sid * _STAGE, _STAGE)],
                    xsh.at[pl.ds(sid * _STAGE, _STAGE)])
    plsc.subcore_barrier()

    base = wid * _CHUNK
    iota = lax.iota(jnp.int32, _L)

    def window(g, carry):
        p = base + g * _W
        pltpu.sync_copy(rows_hbm.at[pl.ds(p, _W)], rows_v)
        pltpu.sync_copy(mask_hbm.at[pl.ds(p, _W)], mask_v)

        def build(j, carry2):
            for k in range(_GI // _L):
                o = j * _GI + k * _L
                rv = rows_v[pl.ds(o, _L)]
                mv = mask_v[pl.ds(o, _L)]
                idx = iota + (p + o) + rv * mv * _F
                idx = jnp.where(idx >= _N, idx - _N, idx)
                idx_v[j, pl.ds(k * _L, _L)] = idx
            return carry2

        lax.fori_loop(0, _NJ, build, 0)

        descs = []
        for j in range(_NJ):
            descs.append(pltpu.async_copy(
                xsh.at[idx_v.at[j]], gout.at[pl.ds(j * _GI, _GI)], sem))
        for d in descs:
            d.wait()

        pltpu.sync_copy(gout, out_hbm.at[pl.ds(p, _W)])
        return carry

    lax.fori_loop(0, _NG, window, 0)


@jax.jit
def _swap_noise(x_flat, mask_i32, rows_flat):
    mesh = plsc.VectorSubcoreMesh(core_axis_name="c", subcore_axis_name="s")
    kern = functools.partial(
        pl.kernel,
        out_type=jax.ShapeDtypeStruct((_N,), jnp.float32),
        mesh=mesh,
        scratch_types=[
            pltpu.VMEM_SHARED((_N,), jnp.float32),
            pltpu.VMEM((_W,), jnp.int32),
            pltpu.VMEM((_W,), jnp.int32),
            pltpu.VMEM((_NJ, _GI), jnp.int32),
            pltpu.VMEM((_W,), jnp.float32),
            pltpu.SemaphoreType.DMA,
        ],
    )(_body)
    return kern(x_flat, mask_i32, rows_flat)


def kernel(x, mask, rows):
    out = _swap_noise(x.reshape(-1),
                      mask.astype(jnp.int32).reshape(-1),
                      rows.reshape(-1))
    return out.reshape(x.shape)


# speedup vs baseline: 1.6576x; 1.3821x over previous
import functools

import jax
import jax.numpy as jnp
from jax import lax
from jax.experimental import pallas as pl
from jax.experimental.pallas import tpu as pltpu
from jax.experimental.pallas import tpu_sc as plsc

_B = 16384
_F = 100
_N = _B * _F

_NC = 2
_NS = 16
_NW = _NC * _NS
_L = 16

_CHUNK = _N // _NW
_W = 3200
_NG = _CHUNK // _W
_GI = 128
_NJ = _W // _GI
_STAGE = _N // _NS


def _body(x_hbm, combo_hbm, out_hbm,
          xsh, cv0, cv1, go0, go1, semc0, semc1, semg, semo0, semo1):
    cid = lax.axis_index("c")
    sid = lax.axis_index("s")
    wid = sid * _NC + cid
    base = wid * _CHUNK
    iota = lax.iota(jnp.int32, _L)

    def combo_in(g, cv):
        return pltpu.make_async_copy(
            combo_hbm.at[pl.ds(base + g * _W, _W)], cv,
            semc0 if cv is cv0 else semc1)

    def out_wr(g, go):
        return pltpu.make_async_copy(
            go, out_hbm.at[pl.ds(base + g * _W, _W)],
            semo0 if go is go0 else semo1)

    combo_in(0, cv0).start()
    pltpu.sync_copy(x_hbm.at[pl.ds(sid * _STAGE, _STAGE)],
                    xsh.at[pl.ds(sid * _STAGE, _STAGE)])
    plsc.subcore_barrier()

    def step(g, cv, go, first, last):
        nxt_cv = cv1 if cv is cv0 else cv0
        if not last:
            combo_in(g + 1, nxt_cv).start()
        combo_in(g, cv).wait()

        p = base + g * _W

        def build(s, carry):
            o = s * _L
            c = cv[pl.ds(o, _L)]
            off = (c >> 1) * (c & 1) * _F
            idx = (p + o) + iota + off
            idx = jnp.where(idx >= _N, idx - _N, idx)
            cv[pl.ds(o, _L)] = idx
            return carry

        lax.fori_loop(0, _W // _L, build, 0)

        if not first:
            out_wr(g - 2, go).wait()
        descs = []
        for j in range(_NJ):
            descs.append(pltpu.async_copy(
                xsh.at[cv.at[pl.ds(j * _GI, _GI)]],
                go.at[pl.ds(j * _GI, _GI)], semg))
        for d in descs:
            d.wait()
        out_wr(g, go).start()

    def pair(h, carry):
        g = h * 2
        step(g, cv0, go0, first=False, last=False)
        step(g + 1, cv1, go1, first=False, last=False)
        return carry

    step(0, cv0, go0, first=True, last=False)
    step(1, cv1, go1, first=True, last=False)
    lax.fori_loop(1, _NG // 2 - 1, pair, 0)
    step(_NG - 2, cv0, go0, first=False, last=False)
    step(_NG - 1, cv1, go1, first=False, last=True)
    out_wr(_NG - 2, go0).wait()
    out_wr(_NG - 1, go1).wait()


@jax.jit
def _swap_noise(x_flat, combo_flat):
    mesh = plsc.VectorSubcoreMesh(core_axis_name="c", subcore_axis_name="s")
    kern = functools.partial(
        pl.kernel,
        out_type=jax.ShapeDtypeStruct((_N,), jnp.float32),
        mesh=mesh,
        scratch_types=[
            pltpu.VMEM_SHARED((_N,), jnp.float32),
            pltpu.VMEM((_W,), jnp.int32),
            pltpu.VMEM((_W,), jnp.int32),
            pltpu.VMEM((_W,), jnp.float32),
            pltpu.VMEM((_W,), jnp.float32),
            pltpu.SemaphoreType.DMA,
            pltpu.SemaphoreType.DMA,
            pltpu.SemaphoreType.DMA,
            pltpu.SemaphoreType.DMA,
            pltpu.SemaphoreType.DMA,
        ],
    )(_body)
    return kern(x_flat, combo_flat)


def kernel(x, mask, rows):
    combo = ((rows << 1) | mask.astype(jnp.int32)).reshape(-1)
    out = _swap_noise(x.reshape(-1), combo)
    return out.reshape(x.shape)


# speedup vs baseline: 1.6864x; 1.0174x over previous
import functools

import jax
import jax.numpy as jnp
from jax import lax
from jax.experimental import pallas as pl
from jax.experimental.pallas import tpu as pltpu
from jax.experimental.pallas import tpu_sc as plsc

_B = 16384
_F = 100
_N = _B * _F

_NC = 2
_NS = 16
_NW = _NC * _NS
_L = 16

_CHUNK = _N // _NW
_W = 3200
_NG = _CHUNK // _W
_GI = 128
_NJ = _W // _GI
_STAGE = _N // _NS


def _body(x_hbm, combo_hbm, out_hbm,
          xsh, cv0, cv1, go0, go1, semc0, semc1, semg, semo0, semo1):
    cid = lax.axis_index("c")
    sid = lax.axis_index("s")
    wid = sid * _NC + cid
    base = wid * _CHUNK
    iota = lax.iota(jnp.int32, _L)

    def combo_in(g, cv):
        return pltpu.make_async_copy(
            combo_hbm.at[pl.ds(base + g * _W, _W)], cv,
            semc0 if cv is cv0 else semc1)

    def out_wr(g, go):
        return pltpu.make_async_copy(
            go, out_hbm.at[pl.ds(base + g * _W, _W)],
            semo0 if go is go0 else semo1)

    combo_in(0, cv0).start()
    pltpu.sync_copy(x_hbm.at[pl.ds(sid * _STAGE, _STAGE)],
                    xsh.at[pl.ds(sid * _STAGE, _STAGE)])
    plsc.subcore_barrier()

    def gath(cv, go):
        return [pltpu.make_async_copy(
            xsh.at[cv.at[pl.ds(j * _GI, _GI)]],
            go.at[pl.ds(j * _GI, _GI)], semg) for j in range(_NJ)]

    def step(g, cv, go, pcv, pgo, first, second, last):
        combo_in(g, cv).wait()

        p = base + g * _W

        def build(s, carry):
            o = s * _L
            c = cv[pl.ds(o, _L)]
            off = (c >> 1) * (c & 1) * _F
            idx = (p + o) + iota + off
            idx = jnp.where(idx >= _N, idx - _N, idx)
            cv[pl.ds(o, _L)] = idx
            return carry

        lax.fori_loop(0, _W // _L, build, 0)

        if not first:
            for d in gath(pcv, pgo):
                d.wait()
            out_wr(g - 1, pgo).start()
        if not last:
            combo_in(g + 1, pcv).start()
        if not (first or second):
            out_wr(g - 2, go).wait()
        for d in gath(cv, go):
            d.start()

    def pair(h, carry):
        g = h * 2
        step(g, cv0, go0, cv1, go1, first=False, second=False, last=False)
        step(g + 1, cv1, go1, cv0, go0, first=False, second=False, last=False)
        return carry

    step(0, cv0, go0, cv1, go1, first=True, second=True, last=False)
    step(1, cv1, go1, cv0, go0, first=False, second=True, last=False)
    lax.fori_loop(1, _NG // 2 - 1, pair, 0)
    step(_NG - 2, cv0, go0, cv1, go1, first=False, second=False, last=False)
    step(_NG - 1, cv1, go1, cv0, go0, first=False, second=False, last=True)
    for d in gath(cv1, go1):
        d.wait()
    out_wr(_NG - 1, go1).start()
    out_wr(_NG - 2, go0).wait()
    out_wr(_NG - 1, go1).wait()


@jax.jit
def _swap_noise(x_flat, combo_flat):
    mesh = plsc.VectorSubcoreMesh(core_axis_name="c", subcore_axis_name="s")
    kern = functools.partial(
        pl.kernel,
        out_type=jax.ShapeDtypeStruct((_N,), jnp.float32),
        mesh=mesh,
        scratch_types=[
            pltpu.VMEM_SHARED((_N,), jnp.float32),
            pltpu.VMEM((_W,), jnp.int32),
            pltpu.VMEM((_W,), jnp.int32),
            pltpu.VMEM((_W,), jnp.float32),
            pltpu.VMEM((_W,), jnp.float32),
            pltpu.SemaphoreType.DMA,
            pltpu.SemaphoreType.DMA,
            pltpu.SemaphoreType.DMA,
            pltpu.SemaphoreType.DMA,
            pltpu.SemaphoreType.DMA,
        ],
    )(_body)
    return kern(x_flat, combo_flat)


def kernel(x, mask, rows):
    combo = ((rows << 1) | mask.astype(jnp.int32)).reshape(-1)
    out = _swap_noise(x.reshape(-1), combo)
    return out.reshape(x.shape)


# speedup vs baseline: 1.7735x; 1.0516x over previous
import functools

import jax
import jax.numpy as jnp
from jax import lax
from jax.experimental import pallas as pl
from jax.experimental.pallas import tpu as pltpu
from jax.experimental.pallas import tpu_sc as plsc

_B = 16384
_F = 100
_N = _B * _F

_NC = 2
_NS = 16
_NW = _NC * _NS
_L = 16

_CHUNK = _N // _NW
_W = 3200
_NG = _CHUNK // _W
_GI = 128
_NJ = _W // _GI
_STAGE = _N // _NS


def _body(x_hbm, combo_hbm, out_hbm,
          xsh, cv0, cv1, go0, go1, semc0, semc1, semg, semo0, semo1):
    cid = lax.axis_index("c")
    sid = lax.axis_index("s")
    wid = sid * _NC + cid
    base = wid * _CHUNK
    iota = lax.iota(jnp.int32, _L)

    def combo_in(g, cv):
        return pltpu.make_async_copy(
            combo_hbm.at[pl.ds(base + g * _W, _W)], cv,
            semc0 if cv is cv0 else semc1)

    def out_wr(g, go):
        return pltpu.make_async_copy(
            go, out_hbm.at[pl.ds(base + g * _W, _W)],
            semo0 if go is go0 else semo1)

    combo_in(0, cv0).start()
    pltpu.sync_copy(x_hbm.at[pl.ds(sid * _STAGE, _STAGE)],
                    xsh.at[pl.ds(sid * _STAGE, _STAGE)])
    plsc.subcore_barrier()

    def gath(cv, go):
        return [pltpu.make_async_copy(
            xsh.at[cv.at[pl.ds(j * _GI, _GI)]],
            go.at[pl.ds(j * _GI, _GI)], semg) for j in range(_NJ)]

    def step(g, cv, go, pcv, pgo, first, second, last):
        combo_in(g, cv).wait()

        p = base + g * _W

        def build(s, carry):
            o = s * _L
            c = cv[pl.ds(o, _L)]
            idx = (p + o) + iota + c
            idx = jnp.where(idx >= _N, idx - _N, idx)
            cv[pl.ds(o, _L)] = idx
            return carry

        lax.fori_loop(0, _W // _L, build, 0)

        if not first:
            for d in gath(pcv, pgo):
                d.wait()
            out_wr(g - 1, pgo).start()
        if not last:
            combo_in(g + 1, pcv).start()
        if not (first or second):
            out_wr(g - 2, go).wait()
        for d in gath(cv, go):
            d.start()

    def pair(h, carry):
        g = h * 2
        step(g, cv0, go0, cv1, go1, first=False, second=False, last=False)
        step(g + 1, cv1, go1, cv0, go0, first=False, second=False, last=False)
        return carry

    step(0, cv0, go0, cv1, go1, first=True, second=True, last=False)
    step(1, cv1, go1, cv0, go0, first=False, second=True, last=False)
    lax.fori_loop(1, _NG // 2 - 1, pair, 0)
    step(_NG - 2, cv0, go0, cv1, go1, first=False, second=False, last=False)
    step(_NG - 1, cv1, go1, cv0, go0, first=False, second=False, last=True)
    for d in gath(cv1, go1):
        d.wait()
    out_wr(_NG - 1, go1).start()
    out_wr(_NG - 2, go0).wait()
    out_wr(_NG - 1, go1).wait()


@jax.jit
def _swap_noise(x_flat, combo_flat):
    mesh = plsc.VectorSubcoreMesh(core_axis_name="c", subcore_axis_name="s")
    kern = functools.partial(
        pl.kernel,
        out_type=jax.ShapeDtypeStruct((_N,), jnp.float32),
        mesh=mesh,
        scratch_types=[
            pltpu.VMEM_SHARED((_N,), jnp.float32),
            pltpu.VMEM((_W,), jnp.int32),
            pltpu.VMEM((_W,), jnp.int32),
            pltpu.VMEM((_W,), jnp.float32),
            pltpu.VMEM((_W,), jnp.float32),
            pltpu.SemaphoreType.DMA,
            pltpu.SemaphoreType.DMA,
            pltpu.SemaphoreType.DMA,
            pltpu.SemaphoreType.DMA,
            pltpu.SemaphoreType.DMA,
        ],
    )(_body)
    return kern(x_flat, combo_flat)


def kernel(x, mask, rows):
    combo = (mask.astype(jnp.int32) * rows * _F).reshape(-1)
    out = _swap_noise(x.reshape(-1), combo)
    return out.reshape(x.shape)


# speedup vs baseline: 1.8176x; 1.0249x over previous
import functools

import jax
import jax.numpy as jnp
from jax import lax
from jax.experimental import pallas as pl
from jax.experimental.pallas import tpu as pltpu
from jax.experimental.pallas import tpu_sc as plsc

_B = 16384
_F = 100
_N = _B * _F

_NC = 2
_NS = 16
_NW = _NC * _NS
_L = 16

_CHUNK = _N // _NW
_W = 6400
_NG = _CHUNK // _W
_GI = 128
_NJ = _W // _GI
_STAGE = _N // _NS


def _body(x_hbm, combo_hbm, out_hbm,
          xsh, cv0, cv1, go0, go1, semc0, semc1, semg, semo0, semo1):
    cid = lax.axis_index("c")
    sid = lax.axis_index("s")
    wid = sid * _NC + cid
    base = wid * _CHUNK
    iota = lax.iota(jnp.int32, _L)

    def combo_in(g, cv):
        return pltpu.make_async_copy(
            combo_hbm.at[pl.ds(base + g * _W, _W)], cv,
            semc0 if cv is cv0 else semc1)

    def out_wr(g, go):
        return pltpu.make_async_copy(
            go, out_hbm.at[pl.ds(base + g * _W, _W)],
            semo0 if go is go0 else semo1)

    combo_in(0, cv0).start()
    pltpu.sync_copy(x_hbm.at[pl.ds(sid * _STAGE, _STAGE)],
                    xsh.at[pl.ds(sid * _STAGE, _STAGE)])
    plsc.subcore_barrier()

    def gath(cv, go):
        return [pltpu.make_async_copy(
            xsh.at[cv.at[pl.ds(j * _GI, _GI)]],
            go.at[pl.ds(j * _GI, _GI)], semg) for j in range(_NJ)]

    def step(g, cv, go, pcv, pgo, first, second, last):
        combo_in(g, cv).wait()

        p = base + g * _W

        def build(s, carry):
            o = s * _L
            c = cv[pl.ds(o, _L)]
            idx = (p + o) + iota + c
            idx = jnp.where(idx >= _N, idx - _N, idx)
            cv[pl.ds(o, _L)] = idx
            return carry

        lax.fori_loop(0, _W // _L, build, 0)

        if not first:
            for d in gath(pcv, pgo):
                d.wait()
            out_wr(g - 1, pgo).start()
        if not last:
            combo_in(g + 1, pcv).start()
        if not (first or second):
            out_wr(g - 2, go).wait()
        for d in gath(cv, go):
            d.start()

    def pair(h, carry):
        g = h * 2
        step(g, cv0, go0, cv1, go1, first=False, second=False, last=False)
        step(g + 1, cv1, go1, cv0, go0, first=False, second=False, last=False)
        return carry

    step(0, cv0, go0, cv1, go1, first=True, second=True, last=False)
    step(1, cv1, go1, cv0, go0, first=False, second=True, last=False)
    lax.fori_loop(1, _NG // 2 - 1, pair, 0)
    step(_NG - 2, cv0, go0, cv1, go1, first=False, second=False, last=False)
    step(_NG - 1, cv1, go1, cv0, go0, first=False, second=False, last=True)
    for d in gath(cv1, go1):
        d.wait()
    out_wr(_NG - 1, go1).start()
    out_wr(_NG - 2, go0).wait()
    out_wr(_NG - 1, go1).wait()


@jax.jit
def _swap_noise(x_flat, combo_flat):
    mesh = plsc.VectorSubcoreMesh(core_axis_name="c", subcore_axis_name="s")
    kern = functools.partial(
        pl.kernel,
        out_type=jax.ShapeDtypeStruct((_N,), jnp.float32),
        mesh=mesh,
        scratch_types=[
            pltpu.VMEM_SHARED((_N,), jnp.float32),
            pltpu.VMEM((_W,), jnp.int32),
            pltpu.VMEM((_W,), jnp.int32),
            pltpu.VMEM((_W,), jnp.float32),
            pltpu.VMEM((_W,), jnp.float32),
            pltpu.SemaphoreType.DMA,
            pltpu.SemaphoreType.DMA,
            pltpu.SemaphoreType.DMA,
            pltpu.SemaphoreType.DMA,
            pltpu.SemaphoreType.DMA,
        ],
    )(_body)
    return kern(x_flat, combo_flat)


def kernel(x, mask, rows):
    combo = (mask.astype(jnp.int32) * rows * _F).reshape(-1)
    out = _swap_noise(x.reshape(-1), combo)
    return out.reshape(x.shape)
